# baseline (speedup 1.0000x reference)
import functools

import jax
import jax.numpy as jnp
from jax import lax
from jax.experimental import pallas as pl
from jax.experimental.pallas import tpu as pltpu
from jax.experimental.pallas import tpu_sc as plsc

_N = 100000
_E = 1600000
_D = 8
_NC = 2
_NS = 16
_NW = _NC * _NS
_C = 2560
_KSUB = _C // 128
_NCHUNKS = _E // _C
_NPAD = 102400
_NPS = _NPAD // _NS
_EB = 4000
_NB = 2048
_NG = 49
_CORE1 = _NPAD // _NB
_MB = 8000


def _lrelu(x):
    return jnp.where(x > 0, x, 0.1 * x)


def _ln(x, g, b):
    m = jnp.mean(x, axis=-1, keepdims=True)
    v = jnp.mean((x - m) ** 2, axis=-1, keepdims=True)
    return (x - m) * lax.rsqrt(v + 1e-5) * g + b


def _mlp2(x, W1, b1, W2, b2):
    return _lrelu(x @ W1 + b1) @ W2 + b2


def _te_body(ea_ref, U1_ref, B1_ref, W2_ref, B2_ref, te_ref):
    x1 = _lrelu(ea_ref[...] @ U1_ref[...] + B1_ref[...])
    for l in range(5):
        te_ref[l] = x1[:, 8 * l:8 * (l + 1)] @ W2_ref[l] + B2_ref[l]


def _te_all(edge_attr, U1, B1, W2, B2):
    grid = _E // _EB
    return pl.pallas_call(
        _te_body,
        grid=(grid,),
        in_specs=[
            pl.BlockSpec((_EB, 7), lambda i: (i, 0)),
            pl.BlockSpec(U1.shape, lambda i: (0, 0)),
            pl.BlockSpec(B1.shape, lambda i: (0, 0)),
            pl.BlockSpec(W2.shape, lambda i: (0, 0, 0)),
            pl.BlockSpec(B2.shape, lambda i: (0, 0, 0)),
        ],
        out_specs=pl.BlockSpec((5, _EB, _D), lambda i: (0, i, 0)),
        out_shape=jax.ShapeDtypeStruct((5, _E, _D), jnp.float32),
    )(edge_attr, U1, B1, W2, B2)


def _pre_body(x_ref, fW1, fb1, fW2, fb2, fg, fbn, nW1, nb1, nW2, nb2,
              h_ref, xt_ref):
    h = _ln(_mlp2(x_ref[...], fW1[...], fb1[...], fW2[...], fb2[...]),
            fg[...], fbn[...])
    h_ref[...] = h
    xt_ref[...] = _mlp2(h, nW1[...], nb1[...], nW2[...], nb2[...])


def _pre(x, fW1, fb1, fW2, fb2, fg, fbn, nW1, nb1, nW2, nb2):
    grid = _NG
    full = lambda a: pl.BlockSpec(a.shape, lambda i: (0,) * a.ndim)
    ws = [fW1, fb1, fW2, fb2, fg, fbn, nW1, nb1, nW2, nb2]
    return pl.pallas_call(
        _pre_body,
        grid=(grid,),
        in_specs=[pl.BlockSpec((_NB, 5), lambda i: (i, 0))] + [full(w) for w in ws],
        out_specs=[pl.BlockSpec((_NB, _D), lambda i: (i, 0))] * 2,
        out_shape=[jax.ShapeDtypeStruct((_N, _D), jnp.float32)] * 2,
    )(x, fW1, fb1, fW2, fb2, fg, fbn, nW1, nb1, nW2, nb2)


def _upd_body(has_lm, h_ref, a0_ref, a1_ref, g, b,
              lW1, lb1, lW2, lb2, lg, lbn, nW1, nb1, nW2, nb2,
              base_ref, xt_ref):
    h = _ln(h_ref[...] + a0_ref[...] + a1_ref[...], g[...], b[...])
    if has_lm:
        h = _ln(_mlp2(h, lW1[...], lb1[...], lW2[...], lb2[...]),
                lg[...], lbn[...])
    base_ref[...] = h
    xt_ref[...] = _mlp2(h, nW1[...], nb1[...], nW2[...], nb2[...])


def _upd(has_lm, h, agg2, g, b, lW1, lb1, lW2, lb2, lg, lbn,
         nW1, nb1, nW2, nb2):
    grid = _NG
    full = lambda a: pl.BlockSpec(a.shape, lambda i: (0,) * a.ndim)
    ws = [g, b, lW1, lb1, lW2, lb2, lg, lbn, nW1, nb1, nW2, nb2]
    nblk = _CORE1
    return pl.pallas_call(
        functools.partial(_upd_body, has_lm),
        grid=(grid,),
        in_specs=[
            pl.BlockSpec((_NB, _D), lambda i: (i, 0)),
            pl.BlockSpec((_NB, _D), lambda i: (i, 0)),
            pl.BlockSpec((_NB, _D), lambda i, n=nblk: (i + n, 0)),
        ] + [full(w) for w in ws],
        out_specs=[pl.BlockSpec((_NB, _D), lambda i: (i, 0))] * 2,
        out_shape=[jax.ShapeDtypeStruct((_N, _D), jnp.float32)] * 2,
    )(h, agg2, agg2, *ws)


def _fin_body(h_ref, a0_ref, a1_ref, g, b, fW1, fb1, fW2, fb2,
              pW1, pb1, pW2, pb2, pW3, pb3, pW4, pb4, out_ref):
    h = _ln(h_ref[...] + a0_ref[...] + a1_ref[...], g[...], b[...])
    hf = _mlp2(h, fW1[...], fb1[...], fW2[...], fb2[...])
    z = _lrelu(hf @ pW1[...] + pb1[...])
    z = _lrelu(z @ pW2[...] + pb2[...])
    z = _lrelu(z @ pW3[...] + pb3[...])
    out_ref[...] = z @ pW4[...] + pb4[...]


def _fin(h, agg2, g, b, fW1, fb1, fW2, fb2,
         pW1, pb1, pW2, pb2, pW3, pb3, pW4, pb4):
    grid = _NG
    full = lambda a: pl.BlockSpec(a.shape, lambda i: (0,) * a.ndim)
    ws = [g, b, fW1, fb1, fW2, fb2, pW1, pb1, pW2, pb2, pW3, pb3, pW4, pb4]
    nblk = _CORE1
    return pl.pallas_call(
        _fin_body,
        grid=(grid,),
        in_specs=[
            pl.BlockSpec((_NB, _D), lambda i: (i, 0)),
            pl.BlockSpec((_NB, _D), lambda i: (i, 0)),
            pl.BlockSpec((_NB, _D), lambda i, n=nblk: (i + n, 0)),
        ] + [full(w) for w in ws],
        out_specs=pl.BlockSpec((_NB, 2), lambda i: (i, 0)),
        out_shape=jax.ShapeDtypeStruct((_N, 2), jnp.float32),
    )(h, agg2, agg2, *ws)


def _gather_pass_build():
    mesh = plsc.VectorSubcoreMesh(core_axis_name="c", subcore_axis_name="s")

    @functools.partial(
        pl.kernel,
        mesh=mesh,
        out_type=jax.ShapeDtypeStruct((_E, _D), jnp.float32),
        scratch_types=[
            pltpu.VMEM((_C,), jnp.int32),
            pltpu.VMEM((_C, _D), jnp.float32),
            pltpu.SemaphoreType.DMA,
        ],
        compiler_params=pltpu.CompilerParams(use_tc_tiling_on_sc=False),
    )
    def gather_pass(xt_hbm, ei_hbm, g_hbm, src_v, rows_v, sem):
        cid = lax.axis_index("c")
        sid = lax.axis_index("s")
        wid = sid * _NC + cid
        nch_w = (_NCHUNKS - wid + _NW - 1) // _NW

        def chunk(t, carry):
            g = wid + t * _NW
            base = g * _C
            pltpu.sync_copy(ei_hbm.at[pl.ds(base, _C)], src_v)
            cps = [
                pltpu.async_copy(
                    xt_hbm.at[src_v.at[pl.ds(k * 128, 128)]],
                    rows_v.at[pl.ds(k * 128, 128)], sem)
                for k in range(_KSUB)
            ]
            for cp in cps:
                cp.wait()
            pltpu.sync_copy(rows_v, g_hbm.at[pl.ds(base, _C)])
            return carry

        lax.fori_loop(0, nch_w, chunk, 0)

    return gather_pass


def _scatter_pass_build():
    mesh = plsc.VectorSubcoreMesh(core_axis_name="c", subcore_axis_name="s")

    @functools.partial(
        pl.kernel,
        mesh=mesh,
        out_type=jax.ShapeDtypeStruct((_NC * _NPAD, _D), jnp.float32),
        scratch_types=[
            pltpu.VMEM((_KSUB, 128), jnp.int32),
            pltpu.VMEM((_C, _D), jnp.float32),
            pltpu.VMEM_SHARED((_NPAD, _D), jnp.float32),
            pltpu.SemaphoreType.DMA,
        ],
        compiler_params=pltpu.CompilerParams(use_tc_tiling_on_sc=False),
    )
    def scatter_pass(msg_hbm, ei_hbm, zeros_hbm, out_hbm,
                     dst_v, rows_v, agg_sh, sem):
        cid = lax.axis_index("c")
        sid = lax.axis_index("s")
        wid = sid * _NC + cid
        row0 = sid * _NPS

        pltpu.sync_copy(zeros_hbm, agg_sh.at[pl.ds(row0, _NPS)])
        plsc.subcore_barrier()

        nch_w = (_NCHUNKS - wid + _NW - 1) // _NW

        def chunk(t, carry):
            g = wid + t * _NW
            base = g * _C
            cps = [
                pltpu.async_copy(
                    ei_hbm.at[pl.ds(_E + base + k * 128, 128)],
                    dst_v.at[k], sem)
                for k in range(_KSUB)
            ]
            for cp in cps:
                cp.wait()
            pltpu.sync_copy(msg_hbm.at[pl.ds(base, _C)], rows_v)
            cps = [
                pltpu.async_copy(
                    rows_v.at[pl.ds(k * 128, 128)],
                    agg_sh.at[dst_v.at[k]], sem, add=True)
                for k in range(_KSUB)
            ]
            for cp in cps:
                cp.wait()
            return carry

        lax.fori_loop(0, nch_w, chunk, 0)
        plsc.subcore_barrier()
        pltpu.sync_copy(agg_sh.at[pl.ds(row0, _NPS)],
                        out_hbm.at[pl.ds(cid * _NPAD + row0, _NPS)])

    return scatter_pass


def _mul_body(g_ref, te_ref, msg_ref):
    msg_ref[...] = g_ref[...] * te_ref[0]


def _mul(g, te_all, layer):
    grid = _E // _MB
    return pl.pallas_call(
        _mul_body,
        grid=(grid,),
        in_specs=[
            pl.BlockSpec((_MB, _D), lambda i: (i, 0)),
            pl.BlockSpec((1, _MB, _D), lambda i, l=layer: (l, i, 0)),
        ],
        out_specs=pl.BlockSpec((_MB, _D), lambda i: (i, 0)),
        out_shape=jax.ShapeDtypeStruct((_E, _D), jnp.float32),
    )(g, te_all)


_sc_cache = {}


def _edge_layer(xt, te_all, layer, ei2d, zeros):
    if not _sc_cache:
        _sc_cache["g"] = _gather_pass_build()
        _sc_cache["s"] = _scatter_pass_build()
    g = _sc_cache["g"](xt, ei2d)
    msg = _mul(g, te_all, layer)
    return _sc_cache["s"](msg, ei2d, zeros)


def kernel(x, edge_index, edge_attr, fm_W1, fm_b1, fm_W2, fm_b2, fm_g, fm_bn,
           lm_W1, lm_b1, lm_W2, lm_b2, lm_g, lm_bn,
           cWe1, cbe1, cWe2, cbe2, cWn1, cbn1, cWn2, cbn2,
           c_res, c_dps, c_g, c_b,
           ft_W1, ft_b1, ft_W2, ft_b2,
           p_W1, p_b1, p_W2, p_b2, p_W3, p_b3, p_W4, p_b4):
    r2 = lambda v: v.reshape(1, -1)
    scale = (c_dps * c_res).reshape(5, 1, 1)

    U1 = cWe1.transpose(1, 0, 2).reshape(7, 5 * _D)
    B1 = cbe1.reshape(1, 5 * _D)
    te_all = _te_all(edge_attr, U1, B1,
                     cWe2 * scale, (cbe2 * scale[:, 0]).reshape(5, 1, _D))

    ei2d = edge_index.reshape(2 * _E)
    zeros = jnp.zeros((_NPS, _D), jnp.float32)

    h, xt = _pre(x, fm_W1, r2(fm_b1), fm_W2, r2(fm_b2), r2(fm_g), r2(fm_bn),
                 cWn1[0], r2(cbn1[0]), cWn2[0], r2(cbn2[0]))

    for i in range(4):
        agg2 = _edge_layer(xt, te_all, i, ei2d, zeros)
        h, xt = _upd(i == 0, h, agg2, r2(c_g[i]), r2(c_b[i]),
                     lm_W1, r2(lm_b1), lm_W2, r2(lm_b2), r2(lm_g), r2(lm_bn),
                     cWn1[i + 1], r2(cbn1[i + 1]), cWn2[i + 1], r2(cbn2[i + 1]))

    agg2 = _edge_layer(xt, te_all, 4, ei2d, zeros)
    return _fin(h, agg2, r2(c_g[4]), r2(c_b[4]),
                ft_W1, r2(ft_b1), ft_W2, r2(ft_b2),
                p_W1, r2(p_b1), p_W2, r2(p_b2),
                p_W3, r2(p_b3), p_W4, r2(p_b4))


# speedup vs baseline: 1.9452x; 1.9452x over previous
import functools

import jax
import jax.numpy as jnp
from jax import lax
from jax.experimental import pallas as pl
from jax.experimental.pallas import tpu as pltpu
from jax.experimental.pallas import tpu_sc as plsc

_N = 100000
_E = 1600000
_D = 8
_NC = 2
_NS = 16
_NW = _NC * _NS
_C = 2560
_KSUB = _C // 128
_NCHUNKS = _E // _C
_NPAD = 102400
_NPS = _NPAD // _NS
_EB = 2560
_NB = 2048
_NG = 49
_CORE1 = _NPAD // _NB
_MB = 2000


def _lrelu(x):
    return jnp.where(x > 0, x, 0.1 * x)


def _ln(x, g, b):
    m = jnp.mean(x, axis=-1, keepdims=True)
    v = jnp.mean((x - m) ** 2, axis=-1, keepdims=True)
    return (x - m) * lax.rsqrt(v + 1e-5) * g + b


def _mlp2(x, W1, b1, W2, b2):
    return _lrelu(x @ W1 + b1) @ W2 + b2


def _te_body(ea_ref, U1_ref, B1_ref, U2_ref, B2_ref, *te_refs):
    x1 = _lrelu(ea_ref[...] @ U1_ref[...] + B1_ref[...])
    t = x1 @ U2_ref[...] + B2_ref[...]
    for l in range(5):
        te_refs[l][...] = t[:, 8 * l:8 * (l + 1)]


def _te_all(edge_attr, U1, B1, U2, B2):
    grid = _E // _EB
    return pl.pallas_call(
        _te_body,
        grid=(grid,),
        in_specs=[
            pl.BlockSpec((_EB, 7), lambda i: (i, 0)),
            pl.BlockSpec(U1.shape, lambda i: (0, 0)),
            pl.BlockSpec(B1.shape, lambda i: (0, 0)),
            pl.BlockSpec(U2.shape, lambda i: (0, 0)),
            pl.BlockSpec(B2.shape, lambda i: (0, 0)),
        ],
        out_specs=[pl.BlockSpec((_EB, _D), lambda i: (i, 0))] * 5,
        out_shape=[jax.ShapeDtypeStruct((_E, _D), jnp.float32)] * 5,
    )(edge_attr, U1, B1, U2, B2)


def _pre_body(x_ref, fW1, fb1, fW2, fb2, fg, fbn, nW1, nb1, nW2, nb2,
              h_ref, xt_ref):
    h = _ln(_mlp2(x_ref[...], fW1[...], fb1[...], fW2[...], fb2[...]),
            fg[...], fbn[...])
    h_ref[...] = h
    xt_ref[...] = _mlp2(h, nW1[...], nb1[...], nW2[...], nb2[...])


def _pre(x, fW1, fb1, fW2, fb2, fg, fbn, nW1, nb1, nW2, nb2):
    full = lambda a: pl.BlockSpec(a.shape, lambda i: (0,) * a.ndim)
    ws = [fW1, fb1, fW2, fb2, fg, fbn, nW1, nb1, nW2, nb2]
    return pl.pallas_call(
        _pre_body,
        grid=(_NG,),
        in_specs=[pl.BlockSpec((_NB, 5), lambda i: (i, 0))] + [full(w) for w in ws],
        out_specs=[pl.BlockSpec((_NB, _D), lambda i: (i, 0))] * 2,
        out_shape=[jax.ShapeDtypeStruct((_N, _D), jnp.float32)] * 2,
    )(x, fW1, fb1, fW2, fb2, fg, fbn, nW1, nb1, nW2, nb2)


def _upd_body(has_lm, h_ref, a0_ref, a1_ref, g, b,
              lW1, lb1, lW2, lb2, lg, lbn, nW1, nb1, nW2, nb2,
              base_ref, xt_ref):
    h = _ln(h_ref[...] + a0_ref[...] + a1_ref[...], g[...], b[...])
    if has_lm:
        h = _ln(_mlp2(h, lW1[...], lb1[...], lW2[...], lb2[...]),
                lg[...], lbn[...])
    base_ref[...] = h
    xt_ref[...] = _mlp2(h, nW1[...], nb1[...], nW2[...], nb2[...])


def _upd(has_lm, h, agg2, g, b, lW1, lb1, lW2, lb2, lg, lbn,
         nW1, nb1, nW2, nb2):
    full = lambda a: pl.BlockSpec(a.shape, lambda i: (0,) * a.ndim)
    ws = [g, b, lW1, lb1, lW2, lb2, lg, lbn, nW1, nb1, nW2, nb2]
    return pl.pallas_call(
        functools.partial(_upd_body, has_lm),
        grid=(_NG,),
        in_specs=[
            pl.BlockSpec((_NB, _D), lambda i: (i, 0)),
            pl.BlockSpec((_NB, _D), lambda i: (i, 0)),
            pl.BlockSpec((_NB, _D), lambda i: (i + _CORE1, 0)),
        ] + [full(w) for w in ws],
        out_specs=[pl.BlockSpec((_NB, _D), lambda i: (i, 0))] * 2,
        out_shape=[jax.ShapeDtypeStruct((_N, _D), jnp.float32)] * 2,
    )(h, agg2, agg2, *ws)


def _fin_body(h_ref, a0_ref, a1_ref, g, b, fW1, fb1, fW2, fb2,
              pW1, pb1, pW2, pb2, pW3, pb3, pW4, pb4, out_ref):
    h = _ln(h_ref[...] + a0_ref[...] + a1_ref[...], g[...], b[...])
    hf = _mlp2(h, fW1[...], fb1[...], fW2[...], fb2[...])
    z = _lrelu(hf @ pW1[...] + pb1[...])
    z = _lrelu(z @ pW2[...] + pb2[...])
    z = _lrelu(z @ pW3[...] + pb3[...])
    out_ref[...] = z @ pW4[...] + pb4[...]


def _fin(h, agg2, g, b, fW1, fb1, fW2, fb2,
         pW1, pb1, pW2, pb2, pW3, pb3, pW4, pb4):
    full = lambda a: pl.BlockSpec(a.shape, lambda i: (0,) * a.ndim)
    ws = [g, b, fW1, fb1, fW2, fb2, pW1, pb1, pW2, pb2, pW3, pb3, pW4, pb4]
    return pl.pallas_call(
        _fin_body,
        grid=(_NG,),
        in_specs=[
            pl.BlockSpec((_NB, _D), lambda i: (i, 0)),
            pl.BlockSpec((_NB, _D), lambda i: (i, 0)),
            pl.BlockSpec((_NB, _D), lambda i: (i + _CORE1, 0)),
        ] + [full(w) for w in ws],
        out_specs=pl.BlockSpec((_NB, 2), lambda i: (i, 0)),
        out_shape=jax.ShapeDtypeStruct((_N, 2), jnp.float32),
    )(h, agg2, agg2, *ws)


def _mul_body(g_ref, te_ref, msg_ref):
    msg_ref[...] = g_ref[...] * te_ref[...]


def _mul(g, te):
    rows = _E * _D // 128
    gf = g.reshape(rows, 128)
    tf = te.reshape(rows, 128)
    out = pl.pallas_call(
        _mul_body,
        grid=(rows // _MB,),
        in_specs=[pl.BlockSpec((_MB, 128), lambda i: (i, 0))] * 2,
        out_specs=pl.BlockSpec((_MB, 128), lambda i: (i, 0)),
        out_shape=jax.ShapeDtypeStruct((rows, 128), jnp.float32),
    )(gf, tf)
    return out.reshape(_E, _D)


def _gather_pass_build():
    mesh = plsc.VectorSubcoreMesh(core_axis_name="c", subcore_axis_name="s")

    @functools.partial(
        pl.kernel,
        mesh=mesh,
        out_type=jax.ShapeDtypeStruct((_E, _D), jnp.float32),
        scratch_types=[
            pltpu.VMEM((_C,), jnp.int32),
            pltpu.VMEM((_C, _D), jnp.float32),
            pltpu.SemaphoreType.DMA,
        ],
        compiler_params=pltpu.CompilerParams(use_tc_tiling_on_sc=False),
    )
    def gather_pass(xt_hbm, ei_hbm, g_hbm, src_v, rows_v, sem):
        cid = lax.axis_index("c")
        sid = lax.axis_index("s")
        wid = sid * _NC + cid
        nch_w = (_NCHUNKS - wid + _NW - 1) // _NW

        def chunk(t, carry):
            g = wid + t * _NW
            base = g * _C
            pltpu.sync_copy(ei_hbm.at[pl.ds(base, _C)], src_v)
            cps = [
                pltpu.async_copy(
                    xt_hbm.at[src_v.at[pl.ds(k * 128, 128)]],
                    rows_v.at[pl.ds(k * 128, 128)], sem)
                for k in range(_KSUB)
            ]
            for cp in cps:
                cp.wait()
            pltpu.sync_copy(rows_v, g_hbm.at[pl.ds(base, _C)])
            return carry

        lax.fori_loop(0, nch_w, chunk, 0)

    return gather_pass


def _scatter_pass_build():
    mesh = plsc.VectorSubcoreMesh(core_axis_name="c", subcore_axis_name="s")

    @functools.partial(
        pl.kernel,
        mesh=mesh,
        out_type=jax.ShapeDtypeStruct((2 * _NPAD, _D), jnp.float32),
        scratch_types=[
            pltpu.VMEM((_KSUB, 128), jnp.int32),
            pltpu.VMEM((_C, _D), jnp.float32),
            pltpu.VMEM_SHARED((_NPAD, _D), jnp.float32),
            pltpu.SemaphoreType.DMA,
        ],
        compiler_params=pltpu.CompilerParams(use_tc_tiling_on_sc=False),
    )
    def scatter_pass(msg_hbm, ei_hbm, zeros_hbm, out_hbm,
                     dst_v, rows_v, agg_sh, sem):
        cid = lax.axis_index("c")
        sid = lax.axis_index("s")
        wid = sid * _NC + cid
        row0 = sid * _NPS

        pltpu.sync_copy(zeros_hbm, agg_sh.at[pl.ds(row0, _NPS)])
        plsc.subcore_barrier()

        nch_w = (_NCHUNKS - wid + _NW - 1) // _NW

        def chunk(t, carry):
            g = wid + t * _NW
            base = g * _C
            cps = [
                pltpu.async_copy(
                    ei_hbm.at[pl.ds(_E + base + k * 128, 128)],
                    dst_v.at[k], sem)
                for k in range(_KSUB)
            ]
            for cp in cps:
                cp.wait()
            pltpu.sync_copy(msg_hbm.at[pl.ds(base, _C)], rows_v)
            cps = [
                pltpu.async_copy(
                    rows_v.at[pl.ds(k * 128, 128)],
                    agg_sh.at[dst_v.at[k]], sem, add=True)
                for k in range(_KSUB)
            ]
            for cp in cps:
                cp.wait()
            return carry

        lax.fori_loop(0, nch_w, chunk, 0)
        plsc.subcore_barrier()
        pltpu.sync_copy(agg_sh.at[pl.ds(row0, _NPS)],
                        out_hbm.at[pl.ds(cid * _NPAD + row0, _NPS)])

    return scatter_pass


_sc_cache = {}


def _edge_layer(xt, te, ei2d, zeros):
    if not _sc_cache:
        _sc_cache["g"] = _gather_pass_build()
        _sc_cache["s"] = _scatter_pass_build()
    g = _sc_cache["g"](xt, ei2d)
    msg = _mul(g, te)
    return _sc_cache["s"](msg, ei2d, zeros)


def kernel(x, edge_index, edge_attr, fm_W1, fm_b1, fm_W2, fm_b2, fm_g, fm_bn,
           lm_W1, lm_b1, lm_W2, lm_b2, lm_g, lm_bn,
           cWe1, cbe1, cWe2, cbe2, cWn1, cbn1, cWn2, cbn2,
           c_res, c_dps, c_g, c_b,
           ft_W1, ft_b1, ft_W2, ft_b2,
           p_W1, p_b1, p_W2, p_b2, p_W3, p_b3, p_W4, p_b4):
    r2 = lambda v: v.reshape(1, -1)
    scale = (c_dps * c_res).reshape(5, 1, 1)

    U1 = cWe1.transpose(1, 0, 2).reshape(7, 5 * _D)
    B1 = cbe1.reshape(1, 5 * _D)
    W2s = cWe2 * scale
    U2 = jax.scipy.linalg.block_diag(*[W2s[l] for l in range(5)])
    B2 = (cbe2 * scale[:, 0]).reshape(1, 5 * _D)
    te_list = _te_all(edge_attr, U1, B1, U2, B2)

    ei2d = edge_index.reshape(2 * _E)
    zeros = jnp.zeros((_NPS, _D), jnp.float32)

    h, xt = _pre(x, fm_W1, r2(fm_b1), fm_W2, r2(fm_b2), r2(fm_g), r2(fm_bn),
                 cWn1[0], r2(cbn1[0]), cWn2[0], r2(cbn2[0]))

    for i in range(4):
        agg2 = _edge_layer(xt, te_list[i], ei2d, zeros)
        h, xt = _upd(i == 0, h, agg2, r2(c_g[i]), r2(c_b[i]),
                     lm_W1, r2(lm_b1), lm_W2, r2(lm_b2), r2(lm_g), r2(lm_bn),
                     cWn1[i + 1], r2(cbn1[i + 1]), cWn2[i + 1], r2(cbn2[i + 1]))

    agg2 = _edge_layer(xt, te_list[4], ei2d, zeros)
    return _fin(h, agg2, r2(c_g[4]), r2(c_b[4]),
                ft_W1, r2(ft_b1), ft_W2, r2(ft_b2),
                p_W1, r2(p_b1), p_W2, r2(p_b2),
                p_W3, r2(p_b3), p_W4, r2(p_b4))


# speedup vs baseline: 4.1119x; 2.1138x over previous
import functools

import jax
import jax.numpy as jnp
from jax import lax
from jax.experimental import pallas as pl
from jax.experimental.pallas import tpu as pltpu
from jax.experimental.pallas import tpu_sc as plsc

_N = 100000
_E = 1600000
_D = 8
_NC = 2
_NS = 16
_NW = _NC * _NS
_C = 2560
_KSUB = _C // 128
_NCHUNKS = _E // _C
_NPAD = 102400
_NPS = _NPAD // _NS
_EB = 2560
_NB = 2048
_NG = 49
_CORE1 = _NPAD // _NB
_MB = 2000


def _lrelu(x):
    return jnp.where(x > 0, x, 0.1 * x)


def _ln(x, g, b):
    m = jnp.mean(x, axis=-1, keepdims=True)
    v = jnp.mean((x - m) ** 2, axis=-1, keepdims=True)
    return (x - m) * lax.rsqrt(v + 1e-5) * g + b


def _mlp2(x, W1, b1, W2, b2):
    return _lrelu(x @ W1 + b1) @ W2 + b2


def _te_body(ea_ref, U1b_ref, B1b_ref, V_ref, B2f_ref, *te_refs):
    x1 = _lrelu(ea_ref[...] @ U1b_ref[...] + B1b_ref[...])
    for l in range(5):
        te_refs[l][...] = x1 @ V_ref[l] + B2f_ref[l]


_TROWS = _E * _D // 128
_TB = 2000


def _te_all(ea16, U1b, B1b, V, B2f):
    grid = _TROWS // _TB
    return pl.pallas_call(
        _te_body,
        grid=(grid,),
        in_specs=[
            pl.BlockSpec((_TB, 112), lambda i: (i, 0)),
            pl.BlockSpec(U1b.shape, lambda i: (0, 0)),
            pl.BlockSpec(B1b.shape, lambda i: (0, 0)),
            pl.BlockSpec(V.shape, lambda i: (0, 0, 0)),
            pl.BlockSpec(B2f.shape, lambda i: (0, 0, 0)),
        ],
        out_specs=[pl.BlockSpec((_TB, 128), lambda i: (i, 0))] * 5,
        out_shape=[jax.ShapeDtypeStruct((_TROWS, 128), jnp.float32)] * 5,
    )(ea16, U1b, B1b, V, B2f)


def _pre_body(x_ref, fW1, fb1, fW2, fb2, fg, fbn, nW1, nb1, nW2, nb2,
              h_ref, xt_ref):
    h = _ln(_mlp2(x_ref[...], fW1[...], fb1[...], fW2[...], fb2[...]),
            fg[...], fbn[...])
    h_ref[...] = h
    xt_ref[...] = _mlp2(h, nW1[...], nb1[...], nW2[...], nb2[...])


def _pre(x, fW1, fb1, fW2, fb2, fg, fbn, nW1, nb1, nW2, nb2):
    full = lambda a: pl.BlockSpec(a.shape, lambda i: (0,) * a.ndim)
    ws = [fW1, fb1, fW2, fb2, fg, fbn, nW1, nb1, nW2, nb2]
    return pl.pallas_call(
        _pre_body,
        grid=(_NG,),
        in_specs=[pl.BlockSpec((_NB, 5), lambda i: (i, 0))] + [full(w) for w in ws],
        out_specs=[pl.BlockSpec((_NB, _D), lambda i: (i, 0))] * 2,
        out_shape=[jax.ShapeDtypeStruct((_N, _D), jnp.float32)] * 2,
    )(x, fW1, fb1, fW2, fb2, fg, fbn, nW1, nb1, nW2, nb2)


def _upd_body(has_lm, h_ref, a0_ref, a1_ref, g, b,
              lW1, lb1, lW2, lb2, lg, lbn, nW1, nb1, nW2, nb2,
              base_ref, xt_ref):
    h = _ln(h_ref[...] + a0_ref[...] + a1_ref[...], g[...], b[...])
    if has_lm:
        h = _ln(_mlp2(h, lW1[...], lb1[...], lW2[...], lb2[...]),
                lg[...], lbn[...])
    base_ref[...] = h
    xt_ref[...] = _mlp2(h, nW1[...], nb1[...], nW2[...], nb2[...])


def _upd(has_lm, h, agg2, g, b, lW1, lb1, lW2, lb2, lg, lbn,
         nW1, nb1, nW2, nb2):
    full = lambda a: pl.BlockSpec(a.shape, lambda i: (0,) * a.ndim)
    ws = [g, b, lW1, lb1, lW2, lb2, lg, lbn, nW1, nb1, nW2, nb2]
    return pl.pallas_call(
        functools.partial(_upd_body, has_lm),
        grid=(_NG,),
        in_specs=[
            pl.BlockSpec((_NB, _D), lambda i: (i, 0)),
            pl.BlockSpec((_NB, _D), lambda i: (i, 0)),
            pl.BlockSpec((_NB, _D), lambda i: (i + _CORE1, 0)),
        ] + [full(w) for w in ws],
        out_specs=[pl.BlockSpec((_NB, _D), lambda i: (i, 0))] * 2,
        out_shape=[jax.ShapeDtypeStruct((_N, _D), jnp.float32)] * 2,
    )(h, agg2, agg2, *ws)


def _fin_body(h_ref, a0_ref, a1_ref, g, b, fW1, fb1, fW2, fb2,
              pW1, pb1, pW2, pb2, pW3, pb3, pW4, pb4, out_ref):
    h = _ln(h_ref[...] + a0_ref[...] + a1_ref[...], g[...], b[...])
    hf = _mlp2(h, fW1[...], fb1[...], fW2[...], fb2[...])
    z = _lrelu(hf @ pW1[...] + pb1[...])
    z = _lrelu(z @ pW2[...] + pb2[...])
    z = _lrelu(z @ pW3[...] + pb3[...])
    out_ref[...] = z @ pW4[...] + pb4[...]


def _fin(h, agg2, g, b, fW1, fb1, fW2, fb2,
         pW1, pb1, pW2, pb2, pW3, pb3, pW4, pb4):
    full = lambda a: pl.BlockSpec(a.shape, lambda i: (0,) * a.ndim)
    ws = [g, b, fW1, fb1, fW2, fb2, pW1, pb1, pW2, pb2, pW3, pb3, pW4, pb4]
    return pl.pallas_call(
        _fin_body,
        grid=(_NG,),
        in_specs=[
            pl.BlockSpec((_NB, _D), lambda i: (i, 0)),
            pl.BlockSpec((_NB, _D), lambda i: (i, 0)),
            pl.BlockSpec((_NB, _D), lambda i: (i + _CORE1, 0)),
        ] + [full(w) for w in ws],
        out_specs=pl.BlockSpec((_NB, 2), lambda i: (i, 0)),
        out_shape=jax.ShapeDtypeStruct((_N, 2), jnp.float32),
    )(h, agg2, agg2, *ws)


def _mul_body(g_ref, te_ref, msg_ref):
    msg_ref[...] = g_ref[...] * te_ref[...]


def _mul(g, te_flat):
    gf = g.reshape(_TROWS, 128)
    out = pl.pallas_call(
        _mul_body,
        grid=(_TROWS // _MB,),
        in_specs=[pl.BlockSpec((_MB, 128), lambda i: (i, 0))] * 2,
        out_specs=pl.BlockSpec((_MB, 128), lambda i: (i, 0)),
        out_shape=jax.ShapeDtypeStruct((_TROWS, 128), jnp.float32),
    )(gf, te_flat)
    return out.reshape(_E, _D)


def _gather_pass_build():
    mesh = plsc.VectorSubcoreMesh(core_axis_name="c", subcore_axis_name="s")

    @functools.partial(
        pl.kernel,
        mesh=mesh,
        out_type=jax.ShapeDtypeStruct((_E, _D), jnp.float32),
        scratch_types=[
            pltpu.VMEM((_C,), jnp.int32),
            pltpu.VMEM((_C, _D), jnp.float32),
            pltpu.SemaphoreType.DMA,
        ],
        compiler_params=pltpu.CompilerParams(use_tc_tiling_on_sc=False),
    )
    def gather_pass(xt_hbm, ei_hbm, g_hbm, src_v, rows_v, sem):
        cid = lax.axis_index("c")
        sid = lax.axis_index("s")
        wid = sid * _NC + cid
        nch_w = (_NCHUNKS - wid + _NW - 1) // _NW

        def chunk(t, carry):
            g = wid + t * _NW
            base = g * _C
            pltpu.sync_copy(ei_hbm.at[pl.ds(base, _C)], src_v)
            cps = [
                pltpu.async_copy(
                    xt_hbm.at[src_v.at[pl.ds(k * 128, 128)]],
                    rows_v.at[pl.ds(k * 128, 128)], sem)
                for k in range(_KSUB)
            ]
            for cp in cps:
                cp.wait()
            pltpu.sync_copy(rows_v, g_hbm.at[pl.ds(base, _C)])
            return carry

        lax.fori_loop(0, nch_w, chunk, 0)

    return gather_pass


def _scatter_pass_build():
    mesh = plsc.VectorSubcoreMesh(core_axis_name="c", subcore_axis_name="s")

    @functools.partial(
        pl.kernel,
        mesh=mesh,
        out_type=jax.ShapeDtypeStruct((2 * _NPAD, _D), jnp.float32),
        scratch_types=[
            pltpu.VMEM((_KSUB, 128), jnp.int32),
            pltpu.VMEM((_C, _D), jnp.float32),
            pltpu.VMEM_SHARED((_NPAD, _D), jnp.float32),
            pltpu.SemaphoreType.DMA,
        ],
        compiler_params=pltpu.CompilerParams(use_tc_tiling_on_sc=False),
    )
    def scatter_pass(msg_hbm, ei_hbm, zeros_hbm, out_hbm,
                     dst_v, rows_v, agg_sh, sem):
        cid = lax.axis_index("c")
        sid = lax.axis_index("s")
        wid = sid * _NC + cid
        row0 = sid * _NPS

        pltpu.sync_copy(zeros_hbm, agg_sh.at[pl.ds(row0, _NPS)])
        plsc.subcore_barrier()

        nch_w = (_NCHUNKS - wid + _NW - 1) // _NW

        def chunk(t, carry):
            g = wid + t * _NW
            base = g * _C
            cps = [
                pltpu.async_copy(
                    ei_hbm.at[pl.ds(_E + base + k * 128, 128)],
                    dst_v.at[k], sem)
                for k in range(_KSUB)
            ]
            for cp in cps:
                cp.wait()
            pltpu.sync_copy(msg_hbm.at[pl.ds(base, _C)], rows_v)
            cps = [
                pltpu.async_copy(
                    rows_v.at[pl.ds(k * 128, 128)],
                    agg_sh.at[dst_v.at[k]], sem, add=True)
                for k in range(_KSUB)
            ]
            for cp in cps:
                cp.wait()
            return carry

        lax.fori_loop(0, nch_w, chunk, 0)
        plsc.subcore_barrier()
        pltpu.sync_copy(agg_sh.at[pl.ds(row0, _NPS)],
                        out_hbm.at[pl.ds(cid * _NPAD + row0, _NPS)])

    return scatter_pass


_sc_cache = {}


def _edge_layer(xt, te, ei2d, zeros):
    if not _sc_cache:
        _sc_cache["g"] = _gather_pass_build()
        _sc_cache["s"] = _scatter_pass_build()
    g = _sc_cache["g"](xt, ei2d)
    msg = _mul(g, te)
    return _sc_cache["s"](msg, ei2d, zeros)


def kernel(x, edge_index, edge_attr, fm_W1, fm_b1, fm_W2, fm_b2, fm_g, fm_bn,
           lm_W1, lm_b1, lm_W2, lm_b2, lm_g, lm_bn,
           cWe1, cbe1, cWe2, cbe2, cWn1, cbn1, cWn2, cbn2,
           c_res, c_dps, c_g, c_b,
           ft_W1, ft_b1, ft_W2, ft_b2,
           p_W1, p_b1, p_W2, p_b2, p_W3, p_b3, p_W4, p_b4):
    r2 = lambda v: v.reshape(1, -1)
    scale = (c_dps * c_res).reshape(5, 1, 1)

    bd = jax.scipy.linalg.block_diag
    U1 = cWe1.transpose(1, 0, 2).reshape(7, 5 * _D)
    B1 = cbe1.reshape(1, 5 * _D)
    W2s = cWe2 * scale
    U2 = bd(*[W2s[l] for l in range(5)])
    B2 = (cbe2 * scale[:, 0]).reshape(5, _D)
    ea16 = edge_attr.reshape(_E // 16, 112)
    U1b = bd(*([U1] * 16))
    B1b = jnp.tile(B1, (1, 16))
    V = jnp.stack([bd(*([U2[:, 8 * l:8 * (l + 1)]] * 16))
                   for l in range(5)])
    B2f = jnp.tile(B2, (1, 16)).reshape(5, 1, 128)
    te_list = _te_all(ea16, U1b, B1b, V, B2f)

    ei2d = edge_index.reshape(2 * _E)
    zeros = jnp.zeros((_NPS, _D), jnp.float32)

    h, xt = _pre(x, fm_W1, r2(fm_b1), fm_W2, r2(fm_b2), r2(fm_g), r2(fm_bn),
                 cWn1[0], r2(cbn1[0]), cWn2[0], r2(cbn2[0]))

    for i in range(4):
        agg2 = _edge_layer(xt, te_list[i], ei2d, zeros)
        h, xt = _upd(i == 0, h, agg2, r2(c_g[i]), r2(c_b[i]),
                     lm_W1, r2(lm_b1), lm_W2, r2(lm_b2), r2(lm_g), r2(lm_bn),
                     cWn1[i + 1], r2(cbn1[i + 1]), cWn2[i + 1], r2(cbn2[i + 1]))

    agg2 = _edge_layer(xt, te_list[4], ei2d, zeros)
    return _fin(h, agg2, r2(c_g[4]), r2(c_b[4]),
                ft_W1, r2(ft_b1), ft_W2, r2(ft_b2),
                p_W1, r2(p_b1), p_W2, r2(p_b2),
                p_W3, r2(p_b3), p_W4, r2(p_b4))


# speedup vs baseline: 5.5229x; 1.3432x over previous
import functools

import jax
import jax.numpy as jnp
from jax import lax
from jax.experimental import pallas as pl
from jax.experimental.pallas import tpu as pltpu
from jax.experimental.pallas import tpu_sc as plsc

_N = 100000
_E = 1600000
_D = 8
_NC = 2
_NS = 16
_NW = _NC * _NS
_C = 2560
_KSUB = _C // 128
_NCHUNKS = _E // _C
_NPAD = 102400
_NPS = _NPAD // _NS
_EB = 2560
_NB = 2048
_NG = 49
_CORE1 = _NPAD // _NB
_MB = 2000


def _lrelu(x):
    return jnp.where(x > 0, x, 0.1 * x)


def _ln(x, g, b):
    m = jnp.mean(x, axis=-1, keepdims=True)
    v = jnp.mean((x - m) ** 2, axis=-1, keepdims=True)
    return (x - m) * lax.rsqrt(v + 1e-5) * g + b


def _mlp2(x, W1, b1, W2, b2):
    return _lrelu(x @ W1 + b1) @ W2 + b2


def _te_body(ea_ref, U1b_ref, B1b_ref, V_ref, B2f_ref, *te_refs):
    x1 = _lrelu(ea_ref[...] @ U1b_ref[...] + B1b_ref[...])
    for l in range(5):
        te_refs[l][...] = x1 @ V_ref[l] + B2f_ref[l]


_TROWS = _E * _D // 128
_TB = 2000


def _te_all(ea16, U1b, B1b, V, B2f):
    grid = _TROWS // _TB
    return pl.pallas_call(
        _te_body,
        grid=(grid,),
        in_specs=[
            pl.BlockSpec((_TB, 112), lambda i: (i, 0)),
            pl.BlockSpec(U1b.shape, lambda i: (0, 0)),
            pl.BlockSpec(B1b.shape, lambda i: (0, 0)),
            pl.BlockSpec(V.shape, lambda i: (0, 0, 0)),
            pl.BlockSpec(B2f.shape, lambda i: (0, 0, 0)),
        ],
        out_specs=[pl.BlockSpec((_TB, 128), lambda i: (i, 0))] * 5,
        out_shape=[jax.ShapeDtypeStruct((_TROWS, 128), jnp.float32)] * 5,
    )(ea16, U1b, B1b, V, B2f)


_XROWS = _N * _D // 128
_ABR = 2 * _NPAD * _D // 128
_XB = 400
_XG = 16
_A1OFF = _NPAD * _D // 128 // _XB


def _dotH(a, b):
    return lax.dot_general(a, b, (((1,), (0,)), ((), ())),
                           precision=lax.Precision.HIGHEST)


def _lnf(x, A8, g, b):
    m = _dotH(x, A8)
    d = x - m
    v = _dotH(d * d, A8)
    return d * lax.rsqrt(v + 1e-5) * g + b


def _pre_body(x_ref, A8, fW1, fb1, fW2, fb2, fg, fbn, nW1, nb1, nW2, nb2,
              h_ref, xt_ref):
    h = _lnf(_mlp2(x_ref[...], fW1[...], fb1[...], fW2[...], fb2[...]),
             A8[...], fg[...], fbn[...])
    h_ref[...] = h
    xt_ref[...] = _mlp2(h, nW1[...], nb1[...], nW2[...], nb2[...])


def _pre(x16, A8, fW1, fb1, fW2, fb2, fg, fbn, nW1, nb1, nW2, nb2):
    full = lambda a: pl.BlockSpec(a.shape, lambda i: (0,) * a.ndim)
    ws = [A8, fW1, fb1, fW2, fb2, fg, fbn, nW1, nb1, nW2, nb2]
    return pl.pallas_call(
        _pre_body,
        grid=(_XG,),
        in_specs=[pl.BlockSpec((_XB, 80), lambda i: (i, 0))]
        + [full(w) for w in ws],
        out_specs=[pl.BlockSpec((_XB, 128), lambda i: (i, 0))] * 2,
        out_shape=[jax.ShapeDtypeStruct((_XROWS, 128), jnp.float32)] * 2,
    )(x16, *ws)


def _upd_body(has_lm, h_ref, a0_ref, a1_ref, A8, g, b,
              lW1, lb1, lW2, lb2, lg, lbn, nW1, nb1, nW2, nb2,
              base_ref, xt_ref):
    h = _lnf(h_ref[...] + a0_ref[...] + a1_ref[...], A8[...], g[...], b[...])
    if has_lm:
        h = _lnf(_mlp2(h, lW1[...], lb1[...], lW2[...], lb2[...]),
                 A8[...], lg[...], lbn[...])
    base_ref[...] = h
    xt_ref[...] = _mlp2(h, nW1[...], nb1[...], nW2[...], nb2[...])


def _upd(has_lm, h, aggf, A8, g, b, lW1, lb1, lW2, lb2, lg, lbn,
         nW1, nb1, nW2, nb2):
    full = lambda a: pl.BlockSpec(a.shape, lambda i: (0,) * a.ndim)
    ws = [A8, g, b, lW1, lb1, lW2, lb2, lg, lbn, nW1, nb1, nW2, nb2]
    return pl.pallas_call(
        functools.partial(_upd_body, has_lm),
        grid=(_XG,),
        in_specs=[
            pl.BlockSpec((_XB, 128), lambda i: (i, 0)),
            pl.BlockSpec((_XB, 128), lambda i: (i, 0)),
            pl.BlockSpec((_XB, 128), lambda i: (i + _A1OFF, 0)),
        ] + [full(w) for w in ws],
        out_specs=[pl.BlockSpec((_XB, 128), lambda i: (i, 0))] * 2,
        out_shape=[jax.ShapeDtypeStruct((_XROWS, 128), jnp.float32)] * 2,
    )(h, aggf, aggf, *ws)


def _fin_body(h_ref, a0_ref, a1_ref, A8, g, b, fW1, fb1, fW2, fb2,
              pW1, pb1, pW2, pb2, pW3, pb3, pW4, pb4, out_ref):
    h = _lnf(h_ref[...] + a0_ref[...] + a1_ref[...], A8[...], g[...], b[...])
    hf = _mlp2(h, fW1[...], fb1[...], fW2[...], fb2[...])
    z = _lrelu(hf @ pW1[...] + pb1[...])
    z = _lrelu(z @ pW2[...] + pb2[...])
    z = _lrelu(z @ pW3[...] + pb3[...])
    out_ref[...] = z @ pW4[...] + pb4[...]


def _fin(h, aggf, A8, g, b, fW1, fb1, fW2, fb2,
         pW1, pb1, pW2, pb2, pW3, pb3, pW4, pb4):
    full = lambda a: pl.BlockSpec(a.shape, lambda i: (0,) * a.ndim)
    ws = [A8, g, b, fW1, fb1, fW2, fb2,
          pW1, pb1, pW2, pb2, pW3, pb3, pW4, pb4]
    return pl.pallas_call(
        _fin_body,
        grid=(_XG,),
        in_specs=[
            pl.BlockSpec((_XB, 128), lambda i: (i, 0)),
            pl.BlockSpec((_XB, 128), lambda i: (i, 0)),
            pl.BlockSpec((_XB, 128), lambda i: (i + _A1OFF, 0)),
        ] + [full(w) for w in ws],
        out_specs=pl.BlockSpec((_XB, 32), lambda i: (i, 0)),
        out_shape=jax.ShapeDtypeStruct((_XROWS, 32), jnp.float32),
    )(h, aggf, aggf, *ws)


def _mul_body(g_ref, te_ref, msg_ref):
    msg_ref[...] = g_ref[...] * te_ref[...]


def _mul(g, te_flat):
    gf = g.reshape(_TROWS, 128)
    out = pl.pallas_call(
        _mul_body,
        grid=(_TROWS // _MB,),
        in_specs=[pl.BlockSpec((_MB, 128), lambda i: (i, 0))] * 2,
        out_specs=pl.BlockSpec((_MB, 128), lambda i: (i, 0)),
        out_shape=jax.ShapeDtypeStruct((_TROWS, 128), jnp.float32),
    )(gf, te_flat)
    return out.reshape(_E, _D)


def _gather_pass_build():
    mesh = plsc.VectorSubcoreMesh(core_axis_name="c", subcore_axis_name="s")

    @functools.partial(
        pl.kernel,
        mesh=mesh,
        out_type=jax.ShapeDtypeStruct((_E, _D), jnp.float32),
        scratch_types=[
            pltpu.VMEM((_C,), jnp.int32),
            pltpu.VMEM((_C, _D), jnp.float32),
            pltpu.SemaphoreType.DMA,
        ],
        compiler_params=pltpu.CompilerParams(use_tc_tiling_on_sc=False),
    )
    def gather_pass(xt_hbm, ei_hbm, g_hbm, src_v, rows_v, sem):
        cid = lax.axis_index("c")
        sid = lax.axis_index("s")
        wid = sid * _NC + cid
        nch_w = (_NCHUNKS - wid + _NW - 1) // _NW

        def chunk(t, carry):
            g = wid + t * _NW
            base = g * _C
            pltpu.sync_copy(ei_hbm.at[pl.ds(base, _C)], src_v)
            cps = [
                pltpu.async_copy(
                    xt_hbm.at[src_v.at[pl.ds(k * 128, 128)]],
                    rows_v.at[pl.ds(k * 128, 128)], sem)
                for k in range(_KSUB)
            ]
            for cp in cps:
                cp.wait()
            pltpu.sync_copy(rows_v, g_hbm.at[pl.ds(base, _C)])
            return carry

        lax.fori_loop(0, nch_w, chunk, 0)

    return gather_pass


def _scatter_pass_build():
    mesh = plsc.VectorSubcoreMesh(core_axis_name="c", subcore_axis_name="s")

    @functools.partial(
        pl.kernel,
        mesh=mesh,
        out_type=jax.ShapeDtypeStruct((2 * _NPAD, _D), jnp.float32),
        scratch_types=[
            pltpu.VMEM((_KSUB, 128), jnp.int32),
            pltpu.VMEM((_C, _D), jnp.float32),
            pltpu.VMEM_SHARED((_NPAD, _D), jnp.float32),
            pltpu.SemaphoreType.DMA,
        ],
        compiler_params=pltpu.CompilerParams(use_tc_tiling_on_sc=False),
    )
    def scatter_pass(msg_hbm, ei_hbm, zeros_hbm, out_hbm,
                     dst_v, rows_v, agg_sh, sem):
        cid = lax.axis_index("c")
        sid = lax.axis_index("s")
        wid = sid * _NC + cid
        row0 = sid * _NPS

        pltpu.sync_copy(zeros_hbm, agg_sh.at[pl.ds(row0, _NPS)])
        plsc.subcore_barrier()

        nch_w = (_NCHUNKS - wid + _NW - 1) // _NW

        def chunk(t, carry):
            g = wid + t * _NW
            base = g * _C
            cps = [
                pltpu.async_copy(
                    ei_hbm.at[pl.ds(_E + base + k * 128, 128)],
                    dst_v.at[k], sem)
                for k in range(_KSUB)
            ]
            for cp in cps:
                cp.wait()
            pltpu.sync_copy(msg_hbm.at[pl.ds(base, _C)], rows_v)
            cps = [
                pltpu.async_copy(
                    rows_v.at[pl.ds(k * 128, 128)],
                    agg_sh.at[dst_v.at[k]], sem, add=True)
                for k in range(_KSUB)
            ]
            for cp in cps:
                cp.wait()
            return carry

        lax.fori_loop(0, nch_w, chunk, 0)
        plsc.subcore_barrier()
        pltpu.sync_copy(agg_sh.at[pl.ds(row0, _NPS)],
                        out_hbm.at[pl.ds(cid * _NPAD + row0, _NPS)])

    return scatter_pass


_sc_cache = {}


def _edge_layer(xt, te, ei2d, zeros):
    if not _sc_cache:
        _sc_cache["g"] = _gather_pass_build()
        _sc_cache["s"] = _scatter_pass_build()
    g = _sc_cache["g"](xt.reshape(_N, _D), ei2d)
    msg = _mul(g, te)
    agg2 = _sc_cache["s"](msg, ei2d, zeros)
    return agg2.reshape(_ABR, 128)


def kernel(x, edge_index, edge_attr, fm_W1, fm_b1, fm_W2, fm_b2, fm_g, fm_bn,
           lm_W1, lm_b1, lm_W2, lm_b2, lm_g, lm_bn,
           cWe1, cbe1, cWe2, cbe2, cWn1, cbn1, cWn2, cbn2,
           c_res, c_dps, c_g, c_b,
           ft_W1, ft_b1, ft_W2, ft_b2,
           p_W1, p_b1, p_W2, p_b2, p_W3, p_b3, p_W4, p_b4):
    r2 = lambda v: v.reshape(1, -1)
    scale = (c_dps * c_res).reshape(5, 1, 1)

    bd = jax.scipy.linalg.block_diag
    U1 = cWe1.transpose(1, 0, 2).reshape(7, 5 * _D)
    B1 = cbe1.reshape(1, 5 * _D)
    W2s = cWe2 * scale
    U2 = bd(*[W2s[l] for l in range(5)])
    B2 = (cbe2 * scale[:, 0]).reshape(5, _D)
    ea16 = edge_attr.reshape(_E // 16, 112)
    U1b = bd(*([U1] * 16))
    B1b = jnp.tile(B1, (1, 16))
    V = jnp.stack([bd(*([U2[:, 8 * l:8 * (l + 1)]] * 16))
                   for l in range(5)])
    B2f = jnp.tile(B2, (1, 16)).reshape(5, 1, 128)
    te_list = _te_all(ea16, U1b, B1b, V, B2f)

    ei2d = edge_index.reshape(2 * _E)
    zeros = jnp.zeros((_NPS, _D), jnp.float32)

    bd16 = lambda W: bd(*([W] * 16))
    t16 = lambda v: jnp.tile(v.reshape(1, -1), (1, 16))
    A8 = bd(*([jnp.full((_D, _D), 1.0 / _D, jnp.float32)] * 16))
    x16 = x.reshape(_N // 16, 80)

    h, xt = _pre(x16, A8,
                 bd16(fm_W1), t16(fm_b1), bd16(fm_W2), t16(fm_b2),
                 t16(fm_g), t16(fm_bn),
                 bd16(cWn1[0]), t16(cbn1[0]), bd16(cWn2[0]), t16(cbn2[0]))

    lws = [bd16(lm_W1), t16(lm_b1), bd16(lm_W2), t16(lm_b2),
           t16(lm_g), t16(lm_bn)]

    for i in range(4):
        aggf = _edge_layer(xt, te_list[i], ei2d, zeros)
        h, xt = _upd(i == 0, h, aggf, A8, t16(c_g[i]), t16(c_b[i]), *lws,
                     bd16(cWn1[i + 1]), t16(cbn1[i + 1]),
                     bd16(cWn2[i + 1]), t16(cbn2[i + 1]))

    aggf = _edge_layer(xt, te_list[4], ei2d, zeros)
    out = _fin(h, aggf, A8, t16(c_g[4]), t16(c_b[4]),
               bd16(ft_W1), t16(ft_b1), bd16(ft_W2), t16(ft_b2),
               bd16(p_W1), t16(p_b1), bd16(p_W2), t16(p_b2),
               bd16(p_W3), t16(p_b3), bd16(p_W4), t16(p_b4))
    return out.reshape(_N, 2)
